# baseline (speedup 1.0000x reference)
import functools

import jax
import jax.numpy as jnp
from jax import lax
from jax.experimental import pallas as pl
from jax.experimental.pallas import tpu as pltpu
from jax.experimental.pallas import tpu_sc as plsc

N, V, DIM = 4096, 200, 64
NC, NS = 2, 16
B_PER_W = 8
N_WORKERS = V // B_PER_W
LANES = 16
BV = 8


def _sc_gather(W2, idx_hi, lo16):
    mesh = plsc.VectorSubcoreMesh(core_axis_name="c", subcore_axis_name="s")

    @functools.partial(
        pl.kernel,
        mesh=mesh,
        out_type=jax.ShapeDtypeStruct((V, DIM), jnp.float32),
        scratch_types=[
            pltpu.VMEM((B_PER_W,), jnp.int32),
            pltpu.VMEM((B_PER_W, LANES), jnp.int32),
            pltpu.VMEM((B_PER_W, 2 * DIM), jnp.float32),
            pltpu.VMEM((B_PER_W, DIM), jnp.float32),
            pltpu.SemaphoreType.DMA,
        ],
    )
    def gather_kernel(w_hbm, idxhi_hbm, lo16_hbm, out_hbm,
                      idx_v, lo_v, rows_v, out_v, sem):
        wid = lax.axis_index("s") * NC + lax.axis_index("c")

        @pl.when(wid < N_WORKERS)
        def _():
            base = wid * B_PER_W
            pltpu.sync_copy(idxhi_hbm.at[pl.ds(base, B_PER_W)], idx_v)
            pltpu.sync_copy(lo16_hbm.at[pl.ds(base, B_PER_W)], lo_v)
            pltpu.async_copy(w_hbm.at[idx_v], rows_v, sem).wait()
            for r in range(B_PER_W):
                m = lo_v[r, :] > 0
                for j in range(DIM // LANES):
                    low = rows_v[r, pl.ds(j * LANES, LANES)]
                    high = rows_v[r, pl.ds(DIM + j * LANES, LANES)]
                    out_v[r, pl.ds(j * LANES, LANES)] = jnp.where(m, high, low)
            pltpu.sync_copy(out_v, out_hbm.at[pl.ds(base, B_PER_W)])

    return gather_kernel(W2, idx_hi, lo16)


def _add_body(x_ref, e_ref, o_ref):
    o_ref[...] = x_ref[...] + e_ref[...]


def _tc_add(xt, e3):
    return pl.pallas_call(
        _add_body,
        grid=(V // BV,),
        in_specs=[
            pl.BlockSpec((BV, DIM, N), lambda i: (i, 0, 0)),
            pl.BlockSpec((BV, DIM, 1), lambda i: (i, 0, 0)),
        ],
        out_specs=pl.BlockSpec((BV, DIM, N), lambda i: (i, 0, 0)),
        out_shape=jax.ShapeDtypeStruct((V, DIM, N), jnp.float32),
    )(xt, e3)


@jax.jit
def kernel(x, node_types, W):
    idx = node_types.reshape(V)
    idx_hi = idx >> 1
    lo16 = jnp.broadcast_to((idx & 1).reshape(V, 1), (V, LANES))
    W2 = jnp.concatenate([W[0::2], W[1::2]], axis=1)
    embeds = _sc_gather(W2, idx_hi, lo16)
    xt = jnp.transpose(x, (1, 2, 0))
    out_t = _tc_add(xt, embeds.reshape(V, DIM, 1))
    return jnp.transpose(out_t, (2, 0, 1))


# speedup vs baseline: 5.3953x; 5.3953x over previous
import functools

import jax
import jax.numpy as jnp
from jax import lax
from jax.experimental import pallas as pl
from jax.experimental.pallas import tpu as pltpu
from jax.experimental.pallas import tpu_sc as plsc

N, V, DIM = 4096, 200, 64
NC, NS = 2, 16
C_PER_W = 2
LANES = 16
BV = 8
SPLIT = 96


def _sc_gather(wt_flat, eidx):
    mesh = plsc.VectorSubcoreMesh(core_axis_name="c", subcore_axis_name="s")

    @functools.partial(
        pl.kernel,
        mesh=mesh,
        out_type=jax.ShapeDtypeStruct((DIM * V,), jnp.float32),
        scratch_types=[
            pltpu.VMEM((SPLIT,), jnp.int32),
            pltpu.VMEM((V - SPLIT,), jnp.int32),
            pltpu.VMEM((SPLIT,), jnp.float32),
            pltpu.VMEM((V - SPLIT,), jnp.float32),
            pltpu.SemaphoreType.DMA,
        ],
    )
    def gather_kernel(wt_hbm, eidx_hbm, out_hbm,
                      idx_a, idx_b, row_a, row_b, sem):
        wid = lax.axis_index("s") * NC + lax.axis_index("c")
        for k in range(C_PER_W):
            base = (wid * C_PER_W + k) * V
            pltpu.sync_copy(eidx_hbm.at[pl.ds(base, SPLIT)], idx_a)
            pltpu.sync_copy(eidx_hbm.at[pl.ds(base + SPLIT, V - SPLIT)], idx_b)
            pltpu.async_copy(wt_hbm.at[idx_a], row_a, sem).wait()
            pltpu.async_copy(wt_hbm.at[idx_b], row_b, sem).wait()
            pltpu.sync_copy(row_a, out_hbm.at[pl.ds(base, SPLIT)])
            pltpu.sync_copy(row_b, out_hbm.at[pl.ds(base + SPLIT, V - SPLIT)])

    return gather_kernel(wt_flat, eidx)


def _add_body(x_ref, e_ref, o_ref):
    o_ref[...] = x_ref[...] + e_ref[...]


def _tc_add(xt, e3):
    return pl.pallas_call(
        _add_body,
        grid=(V // BV,),
        in_specs=[
            pl.BlockSpec((BV, DIM, N), lambda i: (i, 0, 0)),
            pl.BlockSpec((BV, DIM, 1), lambda i: (i, 0, 0)),
        ],
        out_specs=pl.BlockSpec((BV, DIM, N), lambda i: (i, 0, 0)),
        out_shape=jax.ShapeDtypeStruct((V, DIM, N), jnp.float32),
    )(xt, e3)


@jax.jit
def kernel(x, node_types, W):
    idx = node_types.reshape(V)
    eidx = (idx[None, :]
            + 100000 * jnp.arange(DIM, dtype=jnp.int32)[:, None]).reshape(-1)
    wt_flat = jnp.transpose(W).reshape(DIM * 100000)
    e_t = _sc_gather(wt_flat, eidx).reshape(DIM, V)
    embeds = jnp.transpose(e_t)
    xt = jnp.transpose(x, (1, 2, 0))
    out_t = _tc_add(xt, embeds.reshape(V, DIM, 1))
    return jnp.transpose(out_t, (2, 0, 1))


# speedup vs baseline: 5.5552x; 1.0296x over previous
import functools

import jax
import jax.numpy as jnp
from jax import lax
from jax.experimental import pallas as pl
from jax.experimental.pallas import tpu as pltpu
from jax.experimental.pallas import tpu_sc as plsc

N, V, DIM = 4096, 200, 64
NC, NS = 2, 16
C_PER_W = 2
LANES = 16
BV = 8
SPLIT = 96


def _sc_gather(wt_flat, eidx):
    mesh = plsc.VectorSubcoreMesh(core_axis_name="c", subcore_axis_name="s")

    @functools.partial(
        pl.kernel,
        mesh=mesh,
        out_type=jax.ShapeDtypeStruct((DIM * V,), jnp.float32),
        scratch_types=[
            pltpu.VMEM((SPLIT,), jnp.int32),
            pltpu.VMEM((V - SPLIT,), jnp.int32),
            pltpu.VMEM((SPLIT,), jnp.float32),
            pltpu.VMEM((V - SPLIT,), jnp.float32),
            pltpu.SemaphoreType.DMA,
        ],
    )
    def gather_kernel(wt_hbm, eidx_hbm, out_hbm,
                      idx_a, idx_b, row_a, row_b, sem):
        wid = lax.axis_index("s") * NC + lax.axis_index("c")
        for k in range(C_PER_W):
            base = (wid * C_PER_W + k) * V
            pltpu.sync_copy(eidx_hbm.at[pl.ds(base, SPLIT)], idx_a)
            pltpu.sync_copy(eidx_hbm.at[pl.ds(base + SPLIT, V - SPLIT)], idx_b)
            pltpu.async_copy(wt_hbm.at[idx_a], row_a, sem).wait()
            pltpu.async_copy(wt_hbm.at[idx_b], row_b, sem).wait()
            pltpu.sync_copy(row_a, out_hbm.at[pl.ds(base, SPLIT)])
            pltpu.sync_copy(row_b, out_hbm.at[pl.ds(base + SPLIT, V - SPLIT)])

    return gather_kernel(wt_flat, eidx)


def _add_body(x_ref, e_ref, o_ref):
    o_ref[...] = x_ref[...] + e_ref[...][:, :, None]


def _tc_add(xt, e2):
    return pl.pallas_call(
        _add_body,
        grid=(V // BV,),
        in_specs=[
            pl.BlockSpec((BV, DIM, N), lambda i: (i, 0, 0)),
            pl.BlockSpec((BV, DIM), lambda i: (i, 0)),
        ],
        out_specs=pl.BlockSpec((BV, DIM, N), lambda i: (i, 0, 0)),
        out_shape=jax.ShapeDtypeStruct((V, DIM, N), jnp.float32),
    )(xt, e2)


@jax.jit
def kernel(x, node_types, W):
    idx = node_types.reshape(V)
    eidx = (idx[None, :]
            + 100000 * jnp.arange(DIM, dtype=jnp.int32)[:, None]).reshape(-1)
    wt_flat = jnp.transpose(W).reshape(DIM * 100000)
    e_t = _sc_gather(wt_flat, eidx).reshape(DIM, V)
    embeds = jnp.transpose(e_t)
    xt = jnp.transpose(x, (1, 2, 0))
    out_t = _tc_add(xt, embeds)
    return jnp.transpose(out_t, (2, 0, 1))


# speedup vs baseline: 5.6106x; 1.0100x over previous
import functools

import jax
import jax.numpy as jnp
from jax import lax
from jax.experimental import pallas as pl
from jax.experimental.pallas import tpu as pltpu
from jax.experimental.pallas import tpu_sc as plsc

N, V, DIM = 4096, 200, 64
NC, NS = 2, 16
C_PER_W = 2
LANES = 16
BV = 8
SPLIT = 96


def _sc_gather(wt_flat, eidx):
    mesh = plsc.VectorSubcoreMesh(core_axis_name="c", subcore_axis_name="s")

    @functools.partial(
        pl.kernel,
        mesh=mesh,
        out_type=jax.ShapeDtypeStruct((DIM * V,), jnp.float32),
        scratch_types=(
            [pltpu.VMEM((SPLIT,), jnp.int32) for _ in range(C_PER_W)]
            + [pltpu.VMEM((V - SPLIT,), jnp.int32) for _ in range(C_PER_W)]
            + [pltpu.VMEM((SPLIT,), jnp.float32) for _ in range(C_PER_W)]
            + [pltpu.VMEM((V - SPLIT,), jnp.float32) for _ in range(C_PER_W)]
            + [pltpu.SemaphoreType.DMA]
        ),
    )
    def gather_kernel(wt_hbm, eidx_hbm, out_hbm, *scratch):
        idx_a = scratch[0:C_PER_W]
        idx_b = scratch[C_PER_W:2 * C_PER_W]
        row_a = scratch[2 * C_PER_W:3 * C_PER_W]
        row_b = scratch[3 * C_PER_W:4 * C_PER_W]
        sem = scratch[4 * C_PER_W]
        wid = lax.axis_index("s") * NC + lax.axis_index("c")
        for k in range(C_PER_W):
            base = (wid * C_PER_W + k) * V
            pltpu.sync_copy(eidx_hbm.at[pl.ds(base, SPLIT)], idx_a[k])
            pltpu.sync_copy(eidx_hbm.at[pl.ds(base + SPLIT, V - SPLIT)],
                            idx_b[k])
        copies = []
        for k in range(C_PER_W):
            copies.append(pltpu.async_copy(wt_hbm.at[idx_a[k]], row_a[k], sem))
            copies.append(pltpu.async_copy(wt_hbm.at[idx_b[k]], row_b[k], sem))
        for c in copies:
            c.wait()
        for k in range(C_PER_W):
            base = (wid * C_PER_W + k) * V
            pltpu.sync_copy(row_a[k], out_hbm.at[pl.ds(base, SPLIT)])
            pltpu.sync_copy(row_b[k], out_hbm.at[pl.ds(base + SPLIT,
                                                       V - SPLIT)])

    return gather_kernel(wt_flat, eidx)


def _add_body(x_ref, e_ref, o_ref):
    o_ref[...] = x_ref[...] + e_ref[...][:, :, None]


def _tc_add(xt, e2):
    return pl.pallas_call(
        _add_body,
        grid=(V // BV,),
        in_specs=[
            pl.BlockSpec((BV, DIM, N), lambda i: (i, 0, 0)),
            pl.BlockSpec((BV, DIM), lambda i: (i, 0)),
        ],
        out_specs=pl.BlockSpec((BV, DIM, N), lambda i: (i, 0, 0)),
        out_shape=jax.ShapeDtypeStruct((V, DIM, N), jnp.float32),
    )(xt, e2)


@jax.jit
def kernel(x, node_types, W):
    idx = node_types.reshape(V)
    eidx = (idx[None, :]
            + 100000 * jnp.arange(DIM, dtype=jnp.int32)[:, None]).reshape(-1)
    wt_flat = jnp.transpose(W).reshape(DIM * 100000)
    e_t = _sc_gather(wt_flat, eidx).reshape(DIM, V)
    embeds = jnp.transpose(e_t)
    xt = jnp.transpose(x, (1, 2, 0))
    out_t = _tc_add(xt, embeds)
    return jnp.transpose(out_t, (2, 0, 1))
